# baseline (speedup 1.0000x reference)
import functools

import jax
import jax.numpy as jnp
from jax import lax
from jax.experimental import pallas as pl
from jax.experimental.pallas import tpu as pltpu
from jax.experimental.pallas import tpu_sc as plsc

_EMBED = 64
_BATCH = 4096


def _gather_body(uid_hbm, mid_hbm, utab_hbm, mtab_hbm, uout_hbm, mout_hbm,
                 uidx_v, midx_v, urows_v, mrows_v, usem, msem):
    info = plsc.get_sparse_core_info()
    nw = info.num_cores * info.num_subcores
    bpw = _BATCH // nw
    wid = lax.axis_index("s") * info.num_cores + lax.axis_index("c")
    base = wid * bpw
    pltpu.sync_copy(uid_hbm.at[pl.ds(base, bpw)], uidx_v)
    pltpu.sync_copy(mid_hbm.at[pl.ds(base, bpw)], midx_v)
    ucp = pltpu.async_copy(utab_hbm.at[uidx_v], urows_v, usem)
    mcp = pltpu.async_copy(mtab_hbm.at[midx_v], mrows_v, msem)
    ucp.wait()
    pltpu.sync_copy(urows_v, uout_hbm.at[pl.ds(base, bpw)])
    mcp.wait()
    pltpu.sync_copy(mrows_v, mout_hbm.at[pl.ds(base, bpw)])


def _sc_gather(uid, mid, utab, mtab):
    info = plsc.get_sparse_core_info()
    bpw = _BATCH // (info.num_cores * info.num_subcores)
    mesh = plsc.VectorSubcoreMesh(core_axis_name="c", subcore_axis_name="s")
    f = pl.kernel(
        _gather_body,
        out_type=[
            jax.ShapeDtypeStruct((_BATCH, _EMBED), jnp.float32),
            jax.ShapeDtypeStruct((_BATCH, _EMBED), jnp.float32),
        ],
        mesh=mesh,
        scratch_types=[
            pltpu.VMEM((bpw,), jnp.int32),
            pltpu.VMEM((bpw,), jnp.int32),
            pltpu.VMEM((bpw, _EMBED), jnp.float32),
            pltpu.VMEM((bpw, _EMBED), jnp.float32),
            pltpu.SemaphoreType.DMA,
            pltpu.SemaphoreType.DMA,
        ],
        name="sc_embedding_gather",
        compiler_params=pltpu.CompilerParams(use_tc_tiling_on_sc=False),
    )
    return f(uid, mid, utab, mtab)


def _mlp_body(u_ref, m_ref, w1u_ref, w1m_ref, b1_ref, w2_ref, b2_ref,
              w3_ref, b3_ref, out_ref):
    h1 = jnp.dot(u_ref[...], w1u_ref[...], preferred_element_type=jnp.float32)
    h1 = h1 + jnp.dot(m_ref[...], w1m_ref[...], preferred_element_type=jnp.float32)
    h1 = jnp.maximum(h1 + b1_ref[...], 0.0)
    h2 = jnp.dot(h1, w2_ref[...], preferred_element_type=jnp.float32)
    h2 = jnp.maximum(h2 + b2_ref[...], 0.0)
    out_ref[...] = (
        jnp.dot(h2, w3_ref[...], preferred_element_type=jnp.float32) + b3_ref[...]
    )


def _tc_mlp(uemb, memb, W1, b1, W2, b2, W3, b3):
    return pl.pallas_call(
        _mlp_body,
        out_shape=jax.ShapeDtypeStruct((_BATCH, 1), jnp.float32),
    )(
        uemb, memb,
        W1[:_EMBED], W1[_EMBED:],
        b1.reshape(1, -1),
        W2, b2.reshape(1, -1),
        W3, b3.reshape(1, 1),
    )


def kernel(user_id, movie_id, user_table, movie_table, W1, b1, W2, b2, W3, b3):
    uemb, memb = _sc_gather(
        user_id.astype(jnp.int32), movie_id.astype(jnp.int32),
        user_table, movie_table,
    )
    return _tc_mlp(uemb, memb, W1, b1, W2, b2, W3, b3)


# speedup vs baseline: 1.5060x; 1.5060x over previous
import jax
import jax.numpy as jnp
from jax import lax
from jax.experimental import pallas as pl
from jax.experimental.pallas import tpu as pltpu
from jax.experimental.pallas import tpu_sc as plsc

_EMBED = 64
_BATCH = 4096
_VOCAB = 100000
_FULL_TILES = _VOCAB // 128
_TAIL_LO = _FULL_TILES * 128
_TAIL_N = _VOCAB - _TAIL_LO
_W = 1664
_ALEN = 4128
_BS = 32


def _iota16():
    return lax.iota(jnp.int32, 16)


def _gather_body(uid_hbm, mid_hbm, utabt_hbm, mtabt_hbm, tailu_hbm,
                 tailm_hbm, uout_hbm, mout_hbm,
                 idx_v, va, ba, blk, tail_v, rows, bpad, sem):
    info = plsc.get_sparse_core_info()
    nw = info.num_cores * info.num_subcores
    wid = lax.axis_index("s") * info.num_cores + lax.axis_index("c")
    tc0 = (wid * _FULL_TILES) // nw
    tce = ((wid + 1) * _FULL_TILES) // nw
    lo = tc0 * 128
    mid = lo + _W
    hi = jnp.where(wid == nw - 1, _VOCAB, tce * 128)

    for idx_hbm, tail_hbm, out_hbm in (
            (uid_hbm, tailu_hbm, uout_hbm), (mid_hbm, tailm_hbm, mout_hbm)):
        pltpu.sync_copy(idx_hbm, idx_v)
        pltpu.sync_copy(tail_hbm, tail_v)

        def sel(g, cnts):
            c0, c1 = cnts
            idxg = idx_v[pl.ds(g * 16, 16)]
            bvec = g * 16 + _iota16()
            m0 = (idxg >= lo) & (idxg < mid)
            m1 = (idxg >= mid) & (idxg < hi)
            pc0 = jnp.sum(m0.astype(jnp.int32))
            pc1 = jnp.sum(m1.astype(jnp.int32))
            plsc.store_compressed(va.at[pl.ds(c0, 16)], idxg, mask=m0)
            plsc.store_compressed(ba.at[pl.ds(c0, 16)], bvec, mask=m0)
            b1 = _ALEN - c1 - pc1
            plsc.store_compressed(va.at[pl.ds(b1, 16)], idxg, mask=m1)
            plsc.store_compressed(ba.at[pl.ds(b1, 16)], bvec, mask=m1)
            return c0 + pc0, c1 + pc1

        cnt0, cnt1 = lax.fori_loop(0, _BATCH // 16, sel, (0, 0))

        for sw in (0, 1):
            if sw == 0:
                s_sw = pl.multiple_of(lo, 128)
                cnt, jstart0, last = cnt0, 0, cnt0 - 1
            else:
                s_sw = pl.multiple_of(
                    jnp.minimum(lo + _W, _TAIL_LO - _W), 128)
                cnt, jstart0, last = cnt1, _ALEN - cnt1, _ALEN - 1
            pltpu.sync_copy(utabt_hbm.at[:, pl.ds(s_sw, _W)]
                            if out_hbm is uout_hbm else
                            mtabt_hbm.at[:, pl.ds(s_sw, _W)], blk)
            nb = (cnt + _BS - 1) // _BS

            def batch(t, _, jstart0=jstart0, last=last, s_sw=s_sw):
                jstart = jstart0 + t * _BS
                for j in range(_BS):
                    jc = jnp.minimum(jstart + j, last)
                    vsp = plsc.load_gather(va, [jnp.full((16,), jc, jnp.int32)])
                    col = vsp - s_sw
                    mmain = col < _W
                    colc = jnp.minimum(col, _W - 1)
                    trow = jnp.clip(col - _W, 0, _TAIL_N - 1)
                    for k in range(_EMBED // 16):
                        fidx = _iota16() + 16 * k
                        vmain = plsc.load_gather(blk, [fidx, colc])
                        vtail = plsc.load_gather(tail_v, [trow, fidx])
                        rows[j, pl.ds(16 * k, 16)] = jnp.where(
                            mmain, vmain, vtail)
                for g in range(_BS // 16):
                    jjv = jstart + g * 16 + _iota16()
                    jcv = jnp.minimum(jjv, last)
                    bpad[pl.ds(g * 16, 16)] = plsc.load_gather(ba, [jcv])
                pltpu.async_copy(rows, out_hbm.at[bpad], sem).wait()
                return 0

            lax.fori_loop(0, nb, batch, 0)


def _sc_gather(uid, mid, utabt, mtabt, tailu, tailm):
    mesh = plsc.VectorSubcoreMesh(core_axis_name="c", subcore_axis_name="s")
    f = pl.kernel(
        _gather_body,
        out_type=[
            jax.ShapeDtypeStruct((_BATCH, 128), jnp.float32),
            jax.ShapeDtypeStruct((_BATCH, 128), jnp.float32),
        ],
        mesh=mesh,
        scratch_types=[
            pltpu.VMEM((_BATCH,), jnp.int32),
            pltpu.VMEM((_ALEN,), jnp.int32),
            pltpu.VMEM((_ALEN,), jnp.int32),
            pltpu.VMEM((_EMBED, _W), jnp.float32),
            pltpu.VMEM((_TAIL_N, _EMBED), jnp.float32),
            pltpu.VMEM((_BS, 128), jnp.float32),
            pltpu.VMEM((_BS,), jnp.int32),
            pltpu.SemaphoreType.DMA,
        ],
        name="sc_embedding_gather",
        compiler_params=pltpu.CompilerParams(needs_layout_passes=False),
    )
    return f(uid, mid, utabt, mtabt, tailu, tailm)


def _mlp_body(xu_ref, xm_ref, w1u_ref, w1m_ref, b1_ref, w2_ref, b2_ref,
              w3_ref, b3_ref, out_ref):
    u = xu_ref[...][:, :_EMBED]
    m = xm_ref[...][:, :_EMBED]
    h1 = jnp.dot(u, w1u_ref[...], preferred_element_type=jnp.float32)
    h1 = h1 + jnp.dot(m, w1m_ref[...], preferred_element_type=jnp.float32)
    h1 = jnp.maximum(h1 + b1_ref[...], 0.0)
    h2 = jnp.dot(h1, w2_ref[...], preferred_element_type=jnp.float32)
    h2 = jnp.maximum(h2 + b2_ref[...], 0.0)
    out_ref[...] = (
        jnp.dot(h2, w3_ref[...], preferred_element_type=jnp.float32)
        + b3_ref[...]
    )


def _tc_mlp(xu, xm, W1, b1, W2, b2, W3, b3):
    return pl.pallas_call(
        _mlp_body,
        out_shape=jax.ShapeDtypeStruct((_BATCH, 1), jnp.float32),
    )(
        xu, xm,
        W1[:_EMBED], W1[_EMBED:],
        b1.reshape(1, -1),
        W2, b2.reshape(1, -1),
        W3, b3.reshape(1, 1),
    )


def kernel(user_id, movie_id, user_table, movie_table, W1, b1, W2, b2, W3, b3):
    xu, xm = _sc_gather(
        user_id.astype(jnp.int32), movie_id.astype(jnp.int32),
        user_table.T, movie_table.T,
        user_table[_TAIL_LO:], movie_table[_TAIL_LO:],
    )
    return _tc_mlp(xu, xm, W1, b1, W2, b2, W3, b3)


# speedup vs baseline: 1.5894x; 1.0554x over previous
import jax
import jax.numpy as jnp
from jax import lax
from jax.experimental import pallas as pl
from jax.experimental.pallas import tpu as pltpu
from jax.experimental.pallas import tpu_sc as plsc

_EMBED = 64
_BATCH = 4096
_VOCAB = 100000
_FULL_TILES = _VOCAB // 128
_TAIL_LO = _FULL_TILES * 128
_TAIL_N = _VOCAB - _TAIL_LO
_W = 1664
_ALEN = 4128
_BS = 32
_VMASK = (1 << 17) - 1


def _iota16():
    return lax.iota(jnp.int32, 16)


def _extract_entry(va, blk, tail_v, rows, j, jc, s_sw, with_tail):
    vpk = plsc.load_gather(va, [jnp.full((16,), jc, jnp.int32)])
    col = (vpk & _VMASK) - s_sw
    colc = jnp.minimum(col, _W - 1)
    if with_tail:
        mmain = col < _W
        trow = jnp.clip(col - _W, 0, _TAIL_N - 1)
    for k in range(_EMBED // 16):
        fidx = _iota16() + 16 * k
        v = plsc.load_gather(blk, [fidx, colc])
        if with_tail:
            vt = plsc.load_gather(tail_v, [trow, fidx])
            v = jnp.where(mmain, v, vt)
        rows[j, pl.ds(16 * k, 16)] = v


def _run_phase(va, blk, tail_v, rows2, bpad2, out_hbm, ssem,
               cnt, jstart0, last, s_sw, with_tail):
    nb = (cnt + _BS - 1) // _BS
    rows, bpad = rows2[0], bpad2[0]

    def do_batch(t, _):
        jstart = jstart0 + t * _BS
        for j in range(_BS):
            jc = jnp.minimum(jstart + j, last)
            _extract_entry(va, blk, tail_v, rows, j, jc, s_sw, with_tail)
        for g in range(_BS // 16):
            jjv = jstart + g * 16 + _iota16()
            jcv = jnp.minimum(jjv, last)
            bpad[pl.ds(g * 16, 16)] = plsc.load_gather(va, [jcv]) >> 17
        pltpu.async_copy(rows, out_hbm.at[bpad], ssem).wait()
        return 0

    lax.fori_loop(0, nb, do_batch, 0)


def _gather_body(uid_hbm, mid_hbm, utabt_hbm, mtabt_hbm, tailu_hbm,
                 tailm_hbm, uout_hbm, mout_hbm,
                 idx_v, va, blk, tail_v, rows_a, rows_b, bpad_a, bpad_b,
                 dsem, ssem):
    info = plsc.get_sparse_core_info()
    nw = info.num_cores * info.num_subcores
    wid = lax.axis_index("s") * info.num_cores + lax.axis_index("c")
    tc0 = (wid * _FULL_TILES) // nw
    tce = ((wid + 1) * _FULL_TILES) // nw
    lo = tc0 * 128
    mid = lo + _W
    hi = jnp.where(wid == nw - 1, _VOCAB, tce * 128)
    is_last = wid == nw - 1
    rows2 = (rows_a, rows_b)
    bpad2 = (bpad_a, bpad_b)
    s0 = pl.multiple_of(lo, 128)
    s1 = pl.multiple_of(jnp.minimum(lo + _W, _TAIL_LO - _W), 128)

    for idx_hbm, tail_hbm, tabt_hbm, out_hbm in (
            (uid_hbm, tailu_hbm, utabt_hbm, uout_hbm),
            (mid_hbm, tailm_hbm, mtabt_hbm, mout_hbm)):
        cp0 = pltpu.async_copy(tabt_hbm.at[:, pl.ds(s0, _W)], blk, dsem)
        pltpu.sync_copy(idx_hbm, idx_v)
        pltpu.sync_copy(tail_hbm, tail_v)

        def sel(g, cnts):
            c0, c1 = cnts
            idxg = idx_v[pl.ds(g * 16, 16)]
            pk = idxg | ((g * 16 + _iota16()) << 17)
            m0 = (idxg >= lo) & (idxg < mid)
            m1 = (idxg >= mid) & (idxg < hi)
            s = jnp.sum(m0.astype(jnp.int32) + (m1.astype(jnp.int32) << 8))
            pc0 = s & 0xFF
            pc1 = s >> 8
            plsc.store_compressed(va.at[pl.ds(c0, 16)], pk, mask=m0)
            b1 = _ALEN - c1 - pc1
            plsc.store_compressed(va.at[pl.ds(b1, 16)], pk, mask=m1)
            return c0 + pc0, c1 + pc1

        cnt0, cnt1 = lax.fori_loop(0, _BATCH // 16, sel, (0, 0))
        cp0.wait()
        _run_phase(va, blk, tail_v, rows2, bpad2, out_hbm, ssem,
                   cnt0, 0, cnt0 - 1, s0, False)
        cp1 = pltpu.async_copy(tabt_hbm.at[:, pl.ds(s1, _W)], blk, dsem)
        cp1.wait()
        _run_phase(va, blk, tail_v, rows2, bpad2, out_hbm, ssem,
                   cnt1, _ALEN - cnt1, _ALEN - 1, s1, True)


def _sc_gather(uid, mid, utabt, mtabt, tailu, tailm):
    mesh = plsc.VectorSubcoreMesh(core_axis_name="c", subcore_axis_name="s")
    f = pl.kernel(
        _gather_body,
        out_type=[
            jax.ShapeDtypeStruct((_BATCH, 128), jnp.float32),
            jax.ShapeDtypeStruct((_BATCH, 128), jnp.float32),
        ],
        mesh=mesh,
        scratch_types=[
            pltpu.VMEM((_BATCH,), jnp.int32),
            pltpu.VMEM((_ALEN,), jnp.int32),
            pltpu.VMEM((_EMBED, _W), jnp.float32),
            pltpu.VMEM((_TAIL_N, _EMBED), jnp.float32),
            pltpu.VMEM((_BS, 128), jnp.float32),
            pltpu.VMEM((_BS, 128), jnp.float32),
            pltpu.VMEM((_BS,), jnp.int32),
            pltpu.VMEM((_BS,), jnp.int32),
            pltpu.SemaphoreType.DMA,
            pltpu.SemaphoreType.DMA,
        ],
        name="sc_embedding_gather",
        compiler_params=pltpu.CompilerParams(needs_layout_passes=False),
    )
    return f(uid, mid, utabt, mtabt, tailu, tailm)


def _mlp_body(xu_ref, xm_ref, w1u_ref, w1m_ref, b1_ref, w2_ref, b2_ref,
              w3_ref, b3_ref, out_ref):
    u = xu_ref[...][:, :_EMBED]
    m = xm_ref[...][:, :_EMBED]
    h1 = jnp.dot(u, w1u_ref[...], preferred_element_type=jnp.float32)
    h1 = h1 + jnp.dot(m, w1m_ref[...], preferred_element_type=jnp.float32)
    h1 = jnp.maximum(h1 + b1_ref[...], 0.0)
    h2 = jnp.dot(h1, w2_ref[...], preferred_element_type=jnp.float32)
    h2 = jnp.maximum(h2 + b2_ref[...], 0.0)
    out_ref[...] = (
        jnp.dot(h2, w3_ref[...], preferred_element_type=jnp.float32)
        + b3_ref[...]
    )


def _tc_mlp(xu, xm, W1, b1, W2, b2, W3, b3):
    return pl.pallas_call(
        _mlp_body,
        out_shape=jax.ShapeDtypeStruct((_BATCH, 1), jnp.float32),
    )(
        xu, xm,
        W1[:_EMBED], W1[_EMBED:],
        b1.reshape(1, -1),
        W2, b2.reshape(1, -1),
        W3, b3.reshape(1, 1),
    )


def kernel(user_id, movie_id, user_table, movie_table, W1, b1, W2, b2, W3, b3):
    xu, xm = _sc_gather(
        user_id.astype(jnp.int32), movie_id.astype(jnp.int32),
        user_table.T, movie_table.T,
        user_table[_TAIL_LO:], movie_table[_TAIL_LO:],
    )
    return _tc_mlp(xu, xm, W1, b1, W2, b2, W3, b3)
